# Initial kernel scaffold; baseline (speedup 1.0000x reference)
import jax
import jax.numpy as jnp
from jax.experimental import pallas as pl


def kernel(pts_preds, pts_logits, gt_pts):
    raise NotImplementedError("write your pallas kernel here")


# speedup vs baseline: 1.0601x; 1.0601x over previous
import jax
import jax.numpy as jnp
from jax.experimental import pallas as pl


def _body(preds_ref, gt_ref, logits_ref, align_ref, cls_ref):
    align_ref[...] = jnp.sum(jnp.abs(preds_ref[...] - gt_ref[...])).reshape(1, 1)
    x = logits_ref[...]
    sp = jnp.maximum(-x, 0.0) + jnp.log1p(jnp.exp(-jnp.abs(x)))
    cls_ref[...] = jnp.sum(sp).reshape(1, 1)


def kernel(pts_preds, pts_logits, gt_pts):
    B, P, _ = pts_preds.shape
    n = B * P * 2
    preds = pts_preds.reshape(n // 128, 128)
    gt = gt_pts.reshape(n // 128, 128)
    logits = pts_logits.reshape(n // 256, 128)

    s_align, s_cls = pl.pallas_call(
        _body,
        out_shape=(
            jax.ShapeDtypeStruct((1, 1), jnp.float32),
            jax.ShapeDtypeStruct((1, 1), jnp.float32),
        ),
    )(preds, gt, logits)

    pts = jnp.float32(B * P)
    loss_align = s_align[0, 0] / pts
    loss_collinear = jnp.asarray(0.0, jnp.float32)
    loss_cls = 2.0 * s_cls[0, 0] / pts
    dvs = 3.0 * loss_align + loss_collinear + 0.2 * loss_cls
    return (loss_align, loss_collinear, loss_cls, dvs)
